# baseline (speedup 1.0000x reference)
import jax
import jax.numpy as jnp
from jax import lax
from jax.experimental import pallas as pl
from jax.experimental.pallas import tpu as pltpu
from jax.experimental.pallas import tpu_sc as plsc

_ALPHA = 0.5
_EPSILON = 10.0

_NUM_CORES = 2
_NUM_SUBCORES = 16
_LANES = 16
_NW = _NUM_CORES * _NUM_SUBCORES


def _scan_kernel(ps_ref, pt_ref, a_ref):
    ps = ps_ref[...]
    t_dim, n_dim = ps.shape
    x = jnp.concatenate(
        [jnp.ones((t_dim, 1), jnp.float32), 1.0 - ps[:, 1:]], axis=1
    )
    s = 1
    while s < n_dim:
        x = x * jnp.concatenate(
            [jnp.ones((t_dim, s), jnp.float32), x[:, :-s]], axis=1
        )
        s *= 2
    ps_next = jnp.concatenate(
        [ps[:, 1:], jnp.ones((t_dim, 1), jnp.float32)], axis=1
    )
    pt = ps_next * x + _EPSILON / t_dim
    t_col = lax.broadcasted_iota(jnp.int32, (t_dim, n_dim), 0).astype(jnp.float32)
    pt_ref[...] = pt
    a_ref[...] = pt * (1.0 - t_col / t_dim)


def kernel(log_class_probabilities, probability_stopping, y_true):
    T, N, C = log_class_probabilities.shape
    M = T * N
    chunk = M // _NW
    vecs = chunk // _LANES

    pt, a = pl.pallas_call(
        _scan_kernel,
        out_shape=[
            jax.ShapeDtypeStruct((T, N), jnp.float32),
            jax.ShapeDtypeStruct((T, N), jnp.float32),
        ],
    )(probability_stopping)

    pt_flat = jnp.reshape(pt, (-1,))
    w_flat = jnp.reshape(jnp.transpose(a), (-1,))
    yt_flat = jnp.reshape(jnp.transpose(y_true), (-1,))
    lcp_flat = jnp.reshape(log_class_probabilities, (-1,))

    def _sc_body(lcp_hbm, yt_hbm, pt_hbm, w_hbm, out_hbm,
                 y_v, idx_v, g_v, pt_v, w_v, acc_v, sem):
        wid = lax.axis_index("s") * _NUM_CORES + lax.axis_index("c")
        base = wid * chunk
        pltpu.sync_copy(yt_hbm.at[pl.ds(base, chunk)], y_v)
        pltpu.sync_copy(pt_hbm.at[pl.ds(base, chunk)], pt_v)
        pltpu.sync_copy(w_hbm.at[pl.ds(base, chunk)], w_v)

        lanes = lax.iota(jnp.int32, _LANES)

        def build(i, carry):
            j0 = base + i * _LANES
            yv = y_v[pl.ds(i * _LANES, _LANES)]
            idx_v[pl.ds(i * _LANES, _LANES)] = (j0 + lanes) * C + yv
            return carry

        lax.fori_loop(0, vecs, build, 0)

        pltpu.async_copy(lcp_hbm.at[idx_v], g_v, sem).wait()

        def acc(i, carry):
            ac, ae = carry
            gv = g_v[pl.ds(i * _LANES, _LANES)]
            ac = ac + gv * pt_v[pl.ds(i * _LANES, _LANES)]
            ae = ae + jnp.exp(gv) * w_v[pl.ds(i * _LANES, _LANES)]
            return (ac, ae)

        zero = jnp.zeros((_LANES,), jnp.float32)
        ac, ae = lax.fori_loop(0, vecs, acc, (zero, zero))
        acc_v[0, :] = ac
        acc_v[1, :] = ae
        pltpu.sync_copy(acc_v, out_hbm.at[wid])

    sc_call = pl.kernel(
        _sc_body,
        mesh=plsc.VectorSubcoreMesh(core_axis_name="c", subcore_axis_name="s"),
        out_type=jax.ShapeDtypeStruct((_NW, 2, _LANES), jnp.float32),
        scratch_types=[
            pltpu.VMEM((chunk,), jnp.int32),
            pltpu.VMEM((chunk,), jnp.int32),
            pltpu.VMEM((chunk,), jnp.float32),
            pltpu.VMEM((chunk,), jnp.float32),
            pltpu.VMEM((chunk,), jnp.float32),
            pltpu.VMEM((2, _LANES), jnp.float32),
            pltpu.SemaphoreType.DMA,
        ],
    )
    partials = sc_call(lcp_flat, yt_flat, pt_flat, w_flat)

    cls_g = jnp.sum(partials[:, 0, :])
    earl = jnp.sum(partials[:, 1, :])
    return (_ALPHA * (-cls_g) - (1.0 - _ALPHA) * earl) / T
